# Initial kernel scaffold; baseline (speedup 1.0000x reference)
import jax
import jax.numpy as jnp
from jax.experimental import pallas as pl


def kernel(x1, edge_index1, batch1, x2, edge_index2, batch2, target1, target2, cell_expr, params):
    raise NotImplementedError("write your pallas kernel here")


# speedup vs baseline: 11.0259x; 11.0259x over previous
import functools

import jax
import jax.numpy as jnp
from jax import lax
from jax.experimental import pallas as pl
from jax.experimental.pallas import tpu as pltpu
from jax.experimental.pallas import tpu_sc as plsc

N = 10000
E = 320000
D = 128
B = 128
EPS = 1e-5

NC, NS = 2, 16
EPW = E // NS
CH = 128
NCH = EPW // CH
REM = EPW - NCH * CH
RPT = 624
TAIL = N - NS * RPT
ZR = 208

_mesh = plsc.VectorSubcoreMesh(
    core_axis_name="c", subcore_axis_name="s", num_cores=NC, num_subcores=NS)


def _zero_fill(buf, rows, width):
  z = jnp.zeros((16,), jnp.float32)

  def body(i, c):
    for j in range(width // 16):
      buf[i, pl.ds(j * 16, 16)] = z
    return c

  lax.fori_loop(0, rows, body, 0)


@functools.partial(
    pl.kernel,
    out_type=jax.ShapeDtypeStruct((NC, N, D), jnp.float32),
    mesh=_mesh,
    scratch_types=[
        pltpu.VMEM((CH,), jnp.int32),
        pltpu.VMEM((CH,), jnp.int32),
        pltpu.VMEM((REM,), jnp.int32),
        pltpu.VMEM((REM,), jnp.int32),
        pltpu.VMEM((CH, D), jnp.float32),
        pltpu.VMEM((REM, D), jnp.float32),
        pltpu.VMEM((ZR, D), jnp.float32),
        pltpu.VMEM_SHARED((N, D), jnp.float32),
        pltpu.SemaphoreType.DMA,
    ],
)
def _sc_scatter(y_hbm, src_hbm, dst_hbm, out_hbm,
                gidx, sidx, gidx_r, sidx_r, rows, rows_r, zbuf, acc, sem):
  cid = lax.axis_index("c")
  sid = lax.axis_index("s")

  _zero_fill(zbuf, ZR, D)
  for k in range(RPT // ZR):
    pltpu.sync_copy(zbuf, acc.at[pl.ds(sid * RPT + k * ZR, ZR)])

  @pl.when(sid == NS - 1)
  def _():
    pltpu.sync_copy(zbuf.at[pl.ds(0, TAIL)], acc.at[pl.ds(NS * RPT, TAIL)])

  plsc.subcore_barrier()

  ebase = cid * E + sid * EPW

  def do_chunk(off, gi, si, rw, size):
    pltpu.sync_copy(src_hbm.at[pl.ds(off, size)], gi)
    pltpu.async_copy(y_hbm.at[gi], rw, sem).wait()
    pltpu.sync_copy(dst_hbm.at[pl.ds(off, size)], si)
    pltpu.sync_copy(rw, acc.at[si], add=True)

  def body(i, c):
    off = pl.multiple_of(ebase + i * CH, 8)
    do_chunk(off, gidx, sidx, rows, CH)
    return c

  lax.fori_loop(0, NCH, body, 0)
  do_chunk(pl.multiple_of(ebase + NCH * CH, 8), gidx_r, sidx_r, rows_r, REM)

  plsc.subcore_barrier()
  for k in range(RPT // ZR):
    r0 = sid * RPT + k * ZR
    pltpu.sync_copy(acc.at[pl.ds(r0, ZR)], zbuf)
    pltpu.sync_copy(zbuf, out_hbm.at[cid, pl.ds(r0, ZR)])

  @pl.when(sid == NS - 1)
  def _():
    pltpu.sync_copy(acc.at[pl.ds(NS * RPT, TAIL)], zbuf.at[pl.ds(0, TAIL)])
    pltpu.sync_copy(zbuf.at[pl.ds(0, TAIL)],
                    out_hbm.at[cid, pl.ds(NS * RPT, TAIL)])


@functools.partial(
    pl.kernel,
    out_type=jax.ShapeDtypeStruct((NC, N, D), jnp.float32),
    mesh=_mesh,
    scratch_types=[
        pltpu.VMEM((CH,), jnp.int32),
        pltpu.VMEM((REM,), jnp.int32),
        pltpu.VMEM((CH, D), jnp.float32),
        pltpu.VMEM((ZR, D), jnp.float32),
        pltpu.VMEM_SHARED((N, D), jnp.float32),
    ],
)
def _sc_degree(dst_hbm, out_hbm, sidx, sidx_r, ones_v, zbuf, acc):
  cid = lax.axis_index("c")
  sid = lax.axis_index("s")

  _zero_fill(zbuf, ZR, D)
  for k in range(RPT // ZR):
    pltpu.sync_copy(zbuf, acc.at[pl.ds(sid * RPT + k * ZR, ZR)])

  @pl.when(sid == NS - 1)
  def _():
    pltpu.sync_copy(zbuf.at[pl.ds(0, TAIL)], acc.at[pl.ds(NS * RPT, TAIL)])

  one = jnp.ones((16,), jnp.float32)

  def fill(i, c):
    for j in range(D // 16):
      ones_v[i, pl.ds(j * 16, 16)] = one
    return c

  lax.fori_loop(0, CH, fill, 0)
  plsc.subcore_barrier()

  ebase = cid * E + sid * EPW

  def body(i, c):
    off = pl.multiple_of(ebase + i * CH, 8)
    pltpu.sync_copy(dst_hbm.at[pl.ds(off, CH)], sidx)
    pltpu.sync_copy(ones_v, acc.at[sidx], add=True)
    return c

  lax.fori_loop(0, NCH, body, 0)
  offr = pl.multiple_of(ebase + NCH * CH, 8)
  pltpu.sync_copy(dst_hbm.at[pl.ds(offr, REM)], sidx_r)
  pltpu.sync_copy(ones_v.at[pl.ds(0, REM)], acc.at[sidx_r], add=True)

  plsc.subcore_barrier()
  for k in range(RPT // ZR):
    r0 = sid * RPT + k * ZR
    pltpu.sync_copy(acc.at[pl.ds(r0, ZR)], zbuf)
    pltpu.sync_copy(zbuf, out_hbm.at[cid, pl.ds(r0, ZR)])

  @pl.when(sid == NS - 1)
  def _():
    pltpu.sync_copy(acc.at[pl.ds(NS * RPT, TAIL)], zbuf.at[pl.ds(0, TAIL)])
    pltpu.sync_copy(zbuf.at[pl.ds(0, TAIL)],
                    out_hbm.at[cid, pl.ds(NS * RPT, TAIL)])


RB = 1000
_BN_S = 1.0 / (1.0 + EPS) ** 0.5


def _dis(deg_ref):
  return lax.rsqrt(deg_ref[0][:, 0:1] + 1.0)


def _k0_body(deg_ref, x_ref, w_ref, y_ref):
  y_ref[0] = _dis(deg_ref) * jnp.dot(
      x_ref[0], w_ref[0], preferred_element_type=jnp.float32)


def _k12_body(deg_ref, acc_ref, y_ref, w_ref, b_ref, g_ref, bb_ref, ynext_ref):
  dis = _dis(deg_ref)
  out = dis * (acc_ref[0] + y_ref[0]) + b_ref[0]
  h = jnp.maximum(out * (_BN_S * g_ref[0]) + bb_ref[0], 0.0)
  ynext_ref[0] = dis * jnp.dot(h, w_ref[0], preferred_element_type=jnp.float32)


def _k3_body(deg_ref, acc_ref, y_ref, b_ref, x_ref, batch_ref,
             sums_ref, cnt_ref):
  h3 = _dis(deg_ref) * (acc_ref[0] + y_ref[0]) + b_ref[0] + x_ref[0]
  onehot = (batch_ref[0] == lax.broadcasted_iota(jnp.int32, (1, B), 1)
            ).astype(jnp.float32)
  dn = (((0,), (0,)), ((), ()))
  ps = lax.dot_general(onehot, h3, dn, preferred_element_type=jnp.float32)
  pc = lax.dot_general(onehot, jnp.ones((RB, D), jnp.float32), dn,
                       preferred_element_type=jnp.float32)

  @pl.when(pl.program_id(1) == 0)
  def _():
    sums_ref[0] = ps
    cnt_ref[0] = pc

  @pl.when(pl.program_id(1) > 0)
  def _():
    sums_ref[0] += ps
    cnt_ref[0] += pc


def _ln(x, g, b):
  m = jnp.mean(x, axis=-1, keepdims=True)
  v = jnp.mean((x - m) ** 2, axis=-1, keepdims=True)
  return (x - m) * lax.rsqrt(v + EPS) * g + b


def _cls_body(sums_ref, cnt_ref, t1_ref, t2_ref, ce_ref,
              w1_ref, b1_ref, g1_ref, bb1_ref,
              w2_ref, b2_ref, g2_ref, bb2_ref, w3_ref, b3_ref, out_ref):
  d1 = sums_ref[0] / jnp.maximum(cnt_ref[0], 1.0)
  d2 = sums_ref[1] / jnp.maximum(cnt_ref[1], 1.0)
  fused = jnp.concatenate(
      [d1, d2, t1_ref[...], t2_ref[...], ce_ref[...]], axis=1)
  h = jnp.dot(fused, w1_ref[...], preferred_element_type=jnp.float32)
  h = jnp.maximum(_ln(h + b1_ref[...], g1_ref[...], bb1_ref[...]), 0.0)
  h = jnp.dot(h, w2_ref[...], preferred_element_type=jnp.float32)
  h = jnp.maximum(_ln(h + b2_ref[...], g2_ref[...], bb2_ref[...]), 0.0)
  out_ref[...] = jnp.dot(h, w3_ref[...],
                         preferred_element_type=jnp.float32) + b3_ref[...]


def _row_spec(width):
  return pl.BlockSpec((1, RB, width), lambda g, i: (g, i, 0))


_W_SPEC = pl.BlockSpec((1, D, D), lambda g, i: (g, 0, 0))
_V_SPEC = pl.BlockSpec((1, 1, D), lambda g, i: (g, 0, 0))
_GRID = (NC, N // RB)


def _dense_stage0(deg, x, w):
  return pl.pallas_call(
      _k0_body,
      grid=_GRID,
      in_specs=[_row_spec(D), _row_spec(D), _W_SPEC],
      out_specs=_row_spec(D),
      out_shape=jax.ShapeDtypeStruct((NC, N, D), jnp.float32),
  )(deg, x, w)


def _dense_stage12(deg, acc, y, w, b, g, bb):
  return pl.pallas_call(
      _k12_body,
      grid=_GRID,
      in_specs=[_row_spec(D), _row_spec(D), _row_spec(D), _W_SPEC,
                _V_SPEC, _V_SPEC, _V_SPEC],
      out_specs=_row_spec(D),
      out_shape=jax.ShapeDtypeStruct((NC, N, D), jnp.float32),
  )(deg, acc, y, w, b, g, bb)


def _dense_stage3(deg, acc, y, b, x, batch):
  pool_spec = pl.BlockSpec((1, B, D), lambda g, i: (g, 0, 0))
  return pl.pallas_call(
      _k3_body,
      grid=_GRID,
      in_specs=[_row_spec(D), _row_spec(D), _row_spec(D), _V_SPEC,
                _row_spec(D), _row_spec(1)],
      out_specs=[pool_spec, pool_spec],
      out_shape=[jax.ShapeDtypeStruct((NC, B, D), jnp.float32),
                 jax.ShapeDtypeStruct((NC, B, D), jnp.float32)],
  )(deg, acc, y, b, x, batch)


def _classifier(sums, cnt, t1, t2, ce, p):
  w3 = jnp.zeros((D, D), jnp.float32).at[:, :2].set(p['W3'])
  b3 = jnp.zeros((1, D), jnp.float32).at[0, :2].set(p['b3'])
  out = pl.pallas_call(
      _cls_body,
      out_shape=jax.ShapeDtypeStruct((B, D), jnp.float32),
  )(sums, cnt, t1, t2, ce,
    p['W1'], p['b1'].reshape(1, -1), p['ln1_g'].reshape(1, -1),
    p['ln1_b'].reshape(1, -1),
    p['W2'], p['b2'].reshape(1, -1), p['ln2_g'].reshape(1, -1),
    p['ln2_b'].reshape(1, -1), w3, b3)
  return out[:, :2]


def kernel(x1, edge_index1, batch1, x2, edge_index2, batch2,
           target1, target2, cell_expr, params):
  src = jnp.concatenate([edge_index1[0], edge_index2[0] + N])
  dst = jnp.concatenate([edge_index1[1], edge_index2[1]])
  x = jnp.stack([x1, x2])
  batch = jnp.stack([batch1, batch2]).reshape(NC, N, 1)
  p1, p2 = params['g1'], params['g2']

  def stk(name):
    v = jnp.stack([p1[name], p2[name]])
    return v.reshape(NC, 1, D) if v.ndim == 2 else v

  deg = _sc_degree(dst)

  y = _dense_stage0(deg, x, stk('W1'))
  acc = _sc_scatter(y.reshape(NC * N, D), src, dst)
  y = _dense_stage12(deg, acc, y, stk('W2'), stk('b1'), stk('bn1_g'),
                     stk('bn1_b'))
  acc = _sc_scatter(y.reshape(NC * N, D), src, dst)
  y = _dense_stage12(deg, acc, y, stk('W3'), stk('b2'), stk('bn2_g'),
                     stk('bn2_b'))
  acc = _sc_scatter(y.reshape(NC * N, D), src, dst)
  sums, cnt = _dense_stage3(deg, acc, y, stk('b3'), x, batch)

  return _classifier(sums, cnt, target1, target2, cell_expr, params['cls'])
